# Initial kernel scaffold; baseline (speedup 1.0000x reference)
import jax
import jax.numpy as jnp
from jax.experimental import pallas as pl


def kernel(acoustic_input, text_input, speaker_input, embedding_table, speaker_table, W, b):
    raise NotImplementedError("write your pallas kernel here")


# speedup vs baseline: 3.1785x; 3.1785x over previous
import functools

import jax
import jax.numpy as jnp
from jax import lax
from jax.experimental import pallas as pl
from jax.experimental.pallas import tpu as pltpu
from jax.experimental.pallas import tpu_sc as plsc

B, L = 4096, 200
TEXT_DIM, AUDIO_DIM, SPKR_DIM = 128, 64, 32
VOCAB, N_SPK = 1000000, 1000

_INFO = plsc.get_sparse_core_info()
_NC, _NS = _INFO.num_cores, _INFO.num_subcores
_NW = _NC * _NS
_BPW = B // _NW
_LANE = 16
_NCH = _BPW // _LANE
_WIN = 16


def _rowdot_body(x_ref, w_ref, o_ref):
    o_ref[...] = jnp.sum(x_ref[...] * w_ref[...], axis=1, keepdims=True)


def _rowdot(x, w, block_rows):
    n, d = x.shape
    grid = n // block_rows
    return pl.pallas_call(
        _rowdot_body,
        grid=(grid,),
        in_specs=[
            pl.BlockSpec((block_rows, d), lambda i: (i, 0)),
            pl.BlockSpec((1, d), lambda i: (0, 0)),
        ],
        out_specs=pl.BlockSpec((block_rows, 1), lambda i: (i, 0)),
        out_shape=jax.ShapeDtypeStruct((n, 1), jnp.float32),
    )(x, w.reshape(1, d))


def _combine_body(a_ref, t_ref, b_ref, o_ref):
    z = (a_ref[...] + t_ref[...]) * (1.0 / L) + b_ref[...]
    o_ref[...] = jax.nn.sigmoid(z)


def _combine(ac_sum, ts_sum, bias):
    a2 = ac_sum.reshape(32, 128)
    t2 = ts_sum.reshape(32, 128)
    return pl.pallas_call(
        _combine_body,
        out_shape=jax.ShapeDtypeStruct((32, 128), jnp.float32),
    )(a2, t2, bias.reshape(1, 1)).reshape(B)


def _sc_gather_body(text_hbm, spk_hbm, pe_hbm, ps_hbm, out_hbm,
                    idx_t, idx_s, buf_t, buf_s, out_v, sem_t, sem_s):
    wid = lax.axis_index("s") * _NC + lax.axis_index("c")
    b0 = wid * _BPW

    pltpu.sync_copy(text_hbm.at[:, pl.ds(b0, _BPW)], idx_t)
    pltpu.sync_copy(spk_hbm.at[:, pl.ds(b0, _BPW)], idx_s)

    def _fire(l):
        pltpu.async_copy(pe_hbm.at[idx_t.at[l]], buf_t.at[l], sem_t)
        pltpu.async_copy(ps_hbm.at[idx_s.at[l]], buf_s.at[l], sem_s)

    def _drain(l):
        pltpu.make_async_copy(pe_hbm.at[idx_t.at[l]], buf_t.at[l], sem_t).wait()
        pltpu.make_async_copy(ps_hbm.at[idx_s.at[l]], buf_s.at[l], sem_s).wait()

    lax.fori_loop(0, _WIN, lambda l, c: (_fire(l), c)[1], 0)
    lax.fori_loop(_WIN, L, lambda l, c: (_drain(l - _WIN), _fire(l), c)[2], 0)
    lax.fori_loop(L - _WIN, L, lambda l, c: (_drain(l), c)[1], 0)

    for j in range(_NCH):
        o = j * _LANE

        def _acc(l, a):
            return a + buf_t[l, pl.ds(o, _LANE)] + buf_s[l, pl.ds(o, _LANE)]

        out_v[pl.ds(o, _LANE)] = lax.fori_loop(
            0, L, _acc, jnp.zeros((_LANE,), jnp.float32))

    pltpu.sync_copy(out_v, out_hbm.at[pl.ds(b0, _BPW)])


_sc_gather = functools.partial(
    pl.kernel,
    out_type=jax.ShapeDtypeStruct((B,), jnp.float32),
    mesh=plsc.VectorSubcoreMesh(core_axis_name="c", subcore_axis_name="s"),
    scratch_types=[
        pltpu.VMEM((L, _BPW), jnp.int32),
        pltpu.VMEM((L, _BPW), jnp.int32),
        pltpu.VMEM((L, _BPW), jnp.float32),
        pltpu.VMEM((L, _BPW), jnp.float32),
        pltpu.VMEM((_BPW,), jnp.float32),
        pltpu.SemaphoreType.DMA,
        pltpu.SemaphoreType.DMA,
    ],
)(_sc_gather_body)


def kernel(acoustic_input, text_input, speaker_input, embedding_table,
           speaker_table, W, b):
    w_a = W[0, :AUDIO_DIM]
    w_t = W[0, AUDIO_DIM:AUDIO_DIM + TEXT_DIM]
    w_s = W[0, AUDIO_DIM + TEXT_DIM:]

    proj_e = _rowdot(embedding_table, w_t, 8000).reshape(VOCAB)
    proj_s = _rowdot(speaker_table, w_s, N_SPK).reshape(N_SPK)

    ac_flat = acoustic_input.reshape(B, L * AUDIO_DIM)
    w_a_t = jnp.tile(w_a, L)
    ac_sum = _rowdot(ac_flat, w_a_t, 128).reshape(B)

    text_t = jnp.transpose(text_input.astype(jnp.int32), (1, 0))
    spk_t = jnp.transpose(speaker_input.astype(jnp.int32), (1, 0))
    ts_sum = _sc_gather(text_t, spk_t, proj_e, proj_s)

    return _combine(ac_sum, ts_sum, b)
